# baseline (speedup 1.0000x reference)
import functools

import jax
import jax.numpy as jnp
from jax import lax
from jax.experimental import pallas as pl
from jax.experimental.pallas import tpu as pltpu
from jax.experimental.pallas import tpu_sc as plsc

BATCH = 16384
EMBED = 32
NC = 2
NS = 16
NW = NC * NS
B_PER_W = BATCH // NW
CHUNK = 128
NCHUNK = B_PER_W // CHUNK

_mesh = plsc.VectorSubcoreMesh(core_axis_name="c", subcore_axis_name="s")


@functools.partial(
    pl.kernel,
    out_type=jax.ShapeDtypeStruct((BATCH, 2 * EMBED), jnp.float32),
    mesh=_mesh,
    scratch_types=[
        pltpu.VMEM((NCHUNK, CHUNK), jnp.int32),
        pltpu.VMEM((NCHUNK, CHUNK), jnp.int32),
        pltpu.VMEM((B_PER_W, EMBED), jnp.float32),
        pltpu.VMEM((B_PER_W, EMBED), jnp.float32),
        pltpu.SemaphoreType.DMA,
    ],
    compiler_params=pltpu.CompilerParams(use_tc_tiling_on_sc=False),
)
def _lookup_concat(fans_idx, avg_idx, fans_tab, avg_tab, out,
                   fidx_v, aidx_v, fans_v, avg_v, sem):
    wid = lax.axis_index("s") * NC + lax.axis_index("c")
    pltpu.sync_copy(fans_idx.at[wid], fidx_v)
    pltpu.sync_copy(avg_idx.at[wid], aidx_v)
    copies = []
    for j in range(NCHUNK):
        rows = pl.ds(j * CHUNK, CHUNK)
        copies.append(pltpu.async_copy(
            fans_tab.at[fidx_v.at[j]], fans_v.at[rows], sem))
        copies.append(pltpu.async_copy(
            avg_tab.at[aidx_v.at[j]], avg_v.at[rows], sem))
    for c in copies:
        c.wait()
    base = wid * B_PER_W
    out_rows = pl.ds(base, B_PER_W)
    pltpu.sync_copy(fans_v, out.at[out_rows, pl.ds(0, EMBED)])
    pltpu.sync_copy(avg_v, out.at[out_rows, pl.ds(EMBED, EMBED)])


def kernel(fans_idx, avgrating_idx, embedding_fans, embedding_avgrating):
    fidx = fans_idx.astype(jnp.int32).reshape(NW, NCHUNK, CHUNK)
    aidx = avgrating_idx.astype(jnp.int32).reshape(NW, NCHUNK, CHUNK)
    return _lookup_concat(fidx, aidx, embedding_fans, embedding_avgrating)


# speedup vs baseline: 1.2835x; 1.2835x over previous
import functools

import jax
import jax.numpy as jnp
from jax import lax
from jax.experimental import pallas as pl
from jax.experimental.pallas import tpu as pltpu
from jax.experimental.pallas import tpu_sc as plsc

BATCH = 16384
EMBED = 32
NFANS = 1000000
NAVG = 1000
NC = 2
NS = 16
NW = NC * NS
B_PER_W = BATCH // NW
L = 16

NTC = (NFANS + 127) // 128
TC_PER_W = 248
WIN_TC = 4
WIN_COLS = WIN_TC * 128
NWIN = TC_PER_W // WIN_TC
STAGE_CLAMP = (NTC - WIN_TC) * 128

CAP = BATCH + L
OUT_ROWS_F = BATCH + NW
RING = 4

_mesh = plsc.VectorSubcoreMesh(core_axis_name="c", subcore_axis_name="s")


def _row(k_e):
    return jnp.full((L,), k_e, jnp.int32)


@functools.partial(
    pl.kernel,
    out_type=(
        jax.ShapeDtypeStruct((OUT_ROWS_F, 128), jnp.float32),
        jax.ShapeDtypeStruct((BATCH, 128), jnp.float32),
    ),
    mesh=_mesh,
    scratch_types=[
        pltpu.VMEM((BATCH,), jnp.int32),
        pltpu.VMEM((B_PER_W,), jnp.int32),
        pltpu.VMEM((CAP,), jnp.int32),
        pltpu.VMEM((CAP,), jnp.int32),
        [pltpu.VMEM((EMBED, WIN_COLS), jnp.float32) for _ in range(2)],
        pltpu.VMEM((EMBED, 1024), jnp.float32),
        [pltpu.VMEM((L, 128), jnp.float32) for _ in range(RING)],
        pltpu.SMEM((NWIN + 2,), jnp.int32),
        pltpu.SemaphoreType.DMA,
        pltpu.SemaphoreType.DMA,
        pltpu.SemaphoreType.DMA,
    ],
    compiler_params=pltpu.CompilerParams(
        use_tc_tiling_on_sc=True, needs_layout_passes=False),
)
def _lookup(fans_idx, avg_idx, fans_t, avg_t, out_f, out_a,
            fidx_v, aidx_v, blist, blist2, winbufs, avgbuf,
            rows_q, starts, sem_a, sem_b, sem_s):
    wid = lax.axis_index("s") * NC + lax.axis_index("c")
    lo = wid * TC_PER_W
    lane = lax.iota(jnp.int32, L)

    pltpu.sync_copy(fans_idx, fidx_v)
    pltpu.sync_copy(avg_idx.at[pl.ds(wid * B_PER_W, B_PER_W)], aidx_v)
    pltpu.sync_copy(avg_t, avgbuf)

    def scan_body(g, ptr):
        r = fidx_v[pl.ds(g * L, L)]
        j = lax.shift_right_logical(r, 7)
        m = (j >= lo) & (j < lo + TC_PER_W)
        n = plsc.all_reduce_population_count(m)[0]
        @pl.when(n > 0)
        def _():
            plsc.store_compressed(
                blist.at[pl.ds(ptr, L)], g * L + lane, mask=m)
        return ptr + n

    cnt = lax.fori_loop(0, BATCH // L, scan_body, 0)

    starts[0] = 0
    n_groups = lax.div(cnt + L - 1, L)

    def bucket_body(k, ptr2):
        def inner(g, p2):
            b = jnp.clip(blist[pl.ds(g * L, L)], 0, BATCH - 1)
            r = plsc.load_gather(fidx_v, [b])
            wk = lax.shift_right_logical(
                lax.shift_right_logical(r, 7) - lo, 2)
            m = (wk == k) & (g * L + lane < cnt)
            n = plsc.all_reduce_population_count(m)[0]
            @pl.when(n > 0)
            def _():
                plsc.store_compressed(blist2.at[pl.ds(p2, L)], b, mask=m)
            return p2 + n
        ptr2 = lax.fori_loop(0, n_groups, inner, ptr2)
        starts[k + 1] = ptr2
        return ptr2

    lax.fori_loop(0, NWIN, bucket_body, 0)

    def stage_off(k):
        off = jnp.minimum((lo + k * WIN_TC) * 128, STAGE_CLAMP)
        return pl.multiple_of(off, 128)

    def stage_copy(k, buf, sem):
        return pltpu.make_async_copy(
            fans_t.at[:, pl.ds(stage_off(k), WIN_COLS)], buf, sem)

    def process(k, buf):
        s = starts[k]
        e = starts[k + 1]
        stage = stage_off(k)

        def g4body(g4, carry):
            copies = []
            for q in range(RING):
                g = g4 * RING + q
                p = s + g * L + lane
                m = p < e
                b = jnp.clip(
                    plsc.load_gather(blist2, [jnp.minimum(p, cnt - 1)]),
                    0, BATCH - 1)
                r = plsc.load_gather(fidx_v, [b])
                col = jnp.clip(r - stage, 0, WIN_COLS - 1)
                dst = jnp.where(m, b, BATCH + wid)
                for k_e in range(EMBED):
                    v = plsc.load_gather(buf, [_row(k_e), col])
                    plsc.store_scatter(rows_q[q], [lane, _row(k_e)], v)
                copies.append(
                    pltpu.async_copy(rows_q[q], out_f.at[dst], sem_s))
            for c in copies:
                c.wait()
            return carry

        ngr = lax.div(e - s + RING * L - 1, RING * L)
        lax.fori_loop(0, ngr, g4body, 0)

    stage_copy(0, winbufs[0], sem_a).start()

    def pair_body(i, carry):
        k0 = 2 * i
        stage_copy(k0 + 1, winbufs[1], sem_b).start()
        stage_copy(k0, winbufs[0], sem_a).wait()
        process(k0, winbufs[0])
        stage_copy(k0 + 2, winbufs[0], sem_a).start()
        stage_copy(k0 + 1, winbufs[1], sem_b).wait()
        process(k0 + 1, winbufs[1])
        return carry

    lax.fori_loop(0, NWIN // 2, pair_body, 0)
    stage_copy(NWIN, winbufs[0], sem_a).wait()

    def avg_g4(g4, carry):
        copies = []
        for q in range(RING):
            g = g4 * RING + q
            a = aidx_v[pl.ds(g * L, L)]
            dst = wid * B_PER_W + g * L + lane
            for k_e in range(EMBED):
                v = plsc.load_gather(avgbuf, [_row(k_e), a])
                plsc.store_scatter(rows_q[q], [lane, _row(k_e)], v)
            copies.append(
                pltpu.async_copy(rows_q[q], out_a.at[dst], sem_s))
        for c in copies:
            c.wait()
        return carry

    lax.fori_loop(0, B_PER_W // (RING * L), avg_g4, 0)


def kernel(fans_idx, avgrating_idx, embedding_fans, embedding_avgrating):
    avg_p = jnp.pad(embedding_avgrating.astype(jnp.float32).T,
                    ((0, 0), (0, 1024 - NAVG)))
    out_f, out_a = _lookup(
        fans_idx.astype(jnp.int32),
        avgrating_idx.astype(jnp.int32),
        embedding_fans.T,
        avg_p,
    )
    fans_emb = out_f[:BATCH, :EMBED]
    avg_emb = out_a[:, :EMBED]
    return jnp.concatenate((fans_emb, avg_emb), axis=1)


# speedup vs baseline: 3.3286x; 2.5935x over previous
import functools

import jax
import jax.numpy as jnp
from jax import lax
from jax.experimental import pallas as pl
from jax.experimental.pallas import tpu as pltpu
from jax.experimental.pallas import tpu_sc as plsc

BATCH = 16384
EMBED = 32
NFANS = 1000000
NAVG = 1000
NC = 2
NS = 16
NW = NC * NS
B_PER_W = BATCH // NW
L = 16

NTC = (NFANS + 127) // 128
TC_PER_W = 248
WIN_TC = 8
WIN_COLS = WIN_TC * 128
NWIN = TC_PER_W // WIN_TC
STAGE_CLAMP = (NTC - WIN_TC) * 128

CAP = BATCH + L
PAD_ROWS = NW * L
OUT_ROWS_F = BATCH + PAD_ROWS
RING = 4

_mesh = plsc.VectorSubcoreMesh(core_axis_name="c", subcore_axis_name="s")


def _row(k_e):
    return jnp.full((L,), k_e, jnp.int32)


@functools.partial(
    pl.kernel,
    out_type=(
        jax.ShapeDtypeStruct((OUT_ROWS_F, 128), jnp.float32),
        jax.ShapeDtypeStruct((BATCH, 128), jnp.float32),
    ),
    mesh=_mesh,
    scratch_types=[
        pltpu.VMEM((BATCH,), jnp.int32),
        pltpu.VMEM((B_PER_W,), jnp.int32),
        pltpu.VMEM((CAP,), jnp.int32),
        pltpu.VMEM((CAP,), jnp.int32),
        [pltpu.VMEM((EMBED, WIN_COLS), jnp.float32) for _ in range(2)],
        [pltpu.VMEM((L, 128), jnp.float32) for _ in range(RING)],
        pltpu.SMEM((NWIN + 2,), jnp.int32),
        pltpu.SemaphoreType.DMA,
        pltpu.SemaphoreType.DMA,
        pltpu.SemaphoreType.DMA,
    ],
    compiler_params=pltpu.CompilerParams(
        use_tc_tiling_on_sc=True, needs_layout_passes=False),
)
def _lookup(fans_idx, avg_idx, fans_t, avg_t, out_f, out_a,
            fidx_v, aidx_v, blist, blist2, winbufs,
            rows_q, starts, sem_a, sem_b, sem_s):
    wid = lax.axis_index("s") * NC + lax.axis_index("c")
    lo = wid * TC_PER_W
    lane = lax.iota(jnp.int32, L)

    pltpu.sync_copy(fans_idx, fidx_v)
    pltpu.sync_copy(avg_idx.at[pl.ds(wid * B_PER_W, B_PER_W)], aidx_v)

    def scan_body(g, ptr):
        r = fidx_v[pl.ds(g * L, L)]
        j = lax.shift_right_logical(r, 7)
        m = (j >= lo) & (j < lo + TC_PER_W)
        n = plsc.all_reduce_population_count(m)[0]
        @pl.when(n > 0)
        def _():
            plsc.store_compressed(
                blist.at[pl.ds(ptr, L)], g * L + lane, mask=m)
        return ptr + n

    cnt = lax.fori_loop(0, BATCH // L, scan_body, 0)

    starts[0] = 0
    n_groups = lax.div(cnt + L - 1, L)

    def bucket_body(k, ptr2):
        def inner(g, p2):
            b = jnp.clip(blist[pl.ds(g * L, L)], 0, BATCH - 1)
            r = plsc.load_gather(fidx_v, [b])
            wk = lax.shift_right_logical(
                lax.shift_right_logical(r, 7) - lo, 3)
            m = (wk == k) & (g * L + lane < cnt)
            n = plsc.all_reduce_population_count(m)[0]
            @pl.when(n > 0)
            def _():
                plsc.store_compressed(blist2.at[pl.ds(p2, L)], b, mask=m)
            return p2 + n
        ptr2 = lax.fori_loop(0, n_groups, inner, ptr2)
        starts[k + 1] = ptr2
        return ptr2

    lax.fori_loop(0, NWIN, bucket_body, 0)

    def stage_off(k):
        off = jnp.minimum((lo + k * WIN_TC) * 128, STAGE_CLAMP)
        return pl.multiple_of(off, 128)

    def stage_copy(k, buf, sem):
        return pltpu.make_async_copy(
            fans_t.at[:, pl.ds(stage_off(k), WIN_COLS)], buf, sem)

    def process(k, buf):
        s = starts[k]
        e = starts[k + 1]
        stage = stage_off(k)

        @pl.when(e > s)
        def _():
            def gbody(g, carry):
                p = s + g * L + lane
                m = p < e
                b = jnp.clip(
                    plsc.load_gather(blist2, [jnp.minimum(p, cnt - 1)]),
                    0, BATCH - 1)
                r = plsc.load_gather(fidx_v, [b])
                col = jnp.clip(r - stage, 0, WIN_COLS - 1)
                dst = jnp.where(m, b, BATCH + wid * L + lane)
                for k_e in range(EMBED):
                    v = plsc.load_gather(buf, [_row(k_e), col])
                    plsc.store_scatter(rows_q[0], [lane, _row(k_e)], v)
                pltpu.async_copy(rows_q[0], out_f.at[dst], sem_s).wait()
                return carry

            lax.fori_loop(0, lax.div(e - s + L - 1, L), gbody, 0)

    stage_copy(0, winbufs[0], sem_a).start()

    def pair_body(i, carry):
        k0 = 2 * i
        stage_copy(k0 + 1, winbufs[1], sem_b).start()
        stage_copy(k0, winbufs[0], sem_a).wait()
        process(k0, winbufs[0])
        stage_copy(k0 + 2, winbufs[0], sem_a).start()
        stage_copy(k0 + 1, winbufs[1], sem_b).wait()
        process(k0 + 1, winbufs[1])
        return carry

    lax.fori_loop(0, NWIN // 2, pair_body, 0)
    avg_stage = pltpu.make_async_copy(avg_t, winbufs[1], sem_b)
    avg_stage.start()
    stage_copy(NWIN - 1, winbufs[0], sem_a).wait()
    process(NWIN - 1, winbufs[0])
    avg_stage.wait()

    def avg_g4(g4, carry):
        copies = []
        for q in range(RING):
            g = g4 * RING + q
            a = aidx_v[pl.ds(g * L, L)]
            dst = wid * B_PER_W + g * L + lane
            for k_e in range(EMBED):
                v = plsc.load_gather(winbufs[1], [_row(k_e), a])
                plsc.store_scatter(rows_q[q], [lane, _row(k_e)], v)
            copies.append(
                pltpu.async_copy(rows_q[q], out_a.at[dst], sem_s))
        for c in copies:
            c.wait()
        return carry

    lax.fori_loop(0, B_PER_W // (RING * L), avg_g4, 0)


def kernel(fans_idx, avgrating_idx, embedding_fans, embedding_avgrating):
    avg_p = jnp.pad(embedding_avgrating.astype(jnp.float32).T,
                    ((0, 0), (0, 1024 - NAVG)))
    out_f, out_a = _lookup(
        fans_idx.astype(jnp.int32),
        avgrating_idx.astype(jnp.int32),
        embedding_fans.T,
        avg_p,
    )
    fans_emb = out_f[:BATCH, :EMBED]
    avg_emb = out_a[:, :EMBED]
    return jnp.concatenate((fans_emb, avg_emb), axis=1)


# speedup vs baseline: 3.3591x; 1.0092x over previous
import functools

import jax
import jax.numpy as jnp
from jax import lax
from jax.experimental import pallas as pl
from jax.experimental.pallas import tpu as pltpu
from jax.experimental.pallas import tpu_sc as plsc

BATCH = 16384
EMBED = 32
NFANS = 1000000
NAVG = 1000
NC = 2
NS = 16
NW = NC * NS
B_PER_W = BATCH // NW
L = 16

NTC = (NFANS + 127) // 128
TC_PER_W = 248
WIN_TC = 8
WIN_COLS = WIN_TC * 128
NWIN = TC_PER_W // WIN_TC
STAGE_CLAMP = (NTC - WIN_TC) * 128

CAP = BATCH + L
PAD_ROWS = NW * L
OUT_ROWS_F = BATCH + PAD_ROWS
RING = 4

_mesh = plsc.VectorSubcoreMesh(core_axis_name="c", subcore_axis_name="s")


def _row(k_e):
    return jnp.full((L,), k_e, jnp.int32)


@functools.partial(
    pl.kernel,
    out_type=(
        jax.ShapeDtypeStruct((OUT_ROWS_F, 128), jnp.float32),
        jax.ShapeDtypeStruct((BATCH, 128), jnp.float32),
    ),
    mesh=_mesh,
    scratch_types=[
        pltpu.VMEM((BATCH,), jnp.int32),
        pltpu.VMEM((B_PER_W,), jnp.int32),
        pltpu.VMEM((CAP,), jnp.int32),
        pltpu.VMEM((CAP,), jnp.int32),
        [pltpu.VMEM((EMBED, WIN_COLS), jnp.float32) for _ in range(2)],
        [pltpu.VMEM((L, 128), jnp.float32) for _ in range(RING)],
        pltpu.SMEM((NWIN + 2,), jnp.int32),
        pltpu.SemaphoreType.DMA,
        pltpu.SemaphoreType.DMA,
        pltpu.SemaphoreType.DMA,
    ],
    compiler_params=pltpu.CompilerParams(
        use_tc_tiling_on_sc=True, needs_layout_passes=False),
)
def _lookup(fans_idx, avg_idx, fans_t, avg_t, out_f, out_a,
            fidx_v, aidx_v, blist, blist2, winbufs,
            rows_q, starts, sem_a, sem_b, sem_s):
    wid = lax.axis_index("s") * NC + lax.axis_index("c")
    lo = wid * TC_PER_W
    lane = lax.iota(jnp.int32, L)

    pltpu.sync_copy(fans_idx, fidx_v)
    pltpu.sync_copy(avg_idx.at[pl.ds(wid * B_PER_W, B_PER_W)], aidx_v)

    def stage_off(k):
        off = jnp.minimum((lo + k * WIN_TC) * 128, STAGE_CLAMP)
        return pl.multiple_of(off, 128)

    def stage_copy(k, buf, sem):
        return pltpu.make_async_copy(
            fans_t.at[:, pl.ds(stage_off(k), WIN_COLS)], buf, sem)

    stage_copy(0, winbufs[0], sem_a).start()
    stage_copy(1, winbufs[1], sem_b).start()

    def scan_body(g, ptr):
        r = fidx_v[pl.ds(g * L, L)]
        j = lax.shift_right_logical(r, 7)
        m = (j >= lo) & (j < lo + TC_PER_W)
        n = plsc.all_reduce_population_count(m)[0]
        @pl.when(n > 0)
        def _():
            plsc.store_compressed(
                blist.at[pl.ds(ptr, L)], g * L + lane, mask=m)
        return ptr + n

    cnt = lax.fori_loop(0, BATCH // L, scan_body, 0)

    starts[0] = 0
    n_groups = lax.div(cnt + L - 1, L)

    def bucket_body(k, ptr2):
        def inner(g, p2):
            b = jnp.clip(blist[pl.ds(g * L, L)], 0, BATCH - 1)
            r = plsc.load_gather(fidx_v, [b])
            wk = lax.shift_right_logical(
                lax.shift_right_logical(r, 7) - lo, 3)
            m = (wk == k) & (g * L + lane < cnt)
            n = plsc.all_reduce_population_count(m)[0]
            @pl.when(n > 0)
            def _():
                plsc.store_compressed(blist2.at[pl.ds(p2, L)], b, mask=m)
            return p2 + n
        ptr2 = lax.fori_loop(0, n_groups, inner, ptr2)
        starts[k + 1] = ptr2
        return ptr2

    lax.fori_loop(0, NWIN, bucket_body, 0)

    def process(k, buf):
        s = starts[k]
        e = starts[k + 1]
        stage = stage_off(k)

        @pl.when(e > s)
        def _():
            def g2body(g2, carry):
                for q in range(2):
                    g = 2 * g2 + q
                    @pl.when(s + g * L < e)
                    def _(g=g, q=q):
                        p = s + g * L + lane
                        m = p < e
                        b = jnp.clip(
                            plsc.load_gather(
                                blist2, [jnp.minimum(p, cnt - 1)]),
                            0, BATCH - 1)
                        r = plsc.load_gather(fidx_v, [b])
                        col = jnp.clip(r - stage, 0, WIN_COLS - 1)
                        dst = jnp.where(m, b, BATCH + wid * L + lane)
                        for k_e in range(EMBED):
                            v = plsc.load_gather(buf, [_row(k_e), col])
                            plsc.store_scatter(
                                rows_q[q], [lane, _row(k_e)], v)
                        pltpu.make_async_copy(
                            rows_q[q], out_f.at[dst], sem_s).start()
                for q in range(2):
                    g = 2 * g2 + q
                    @pl.when(s + g * L < e)
                    def _(q=q):
                        pltpu.make_async_copy(
                            rows_q[q], out_f.at[lane], sem_s).wait()
                return carry

            lax.fori_loop(0, lax.div(e - s + 2 * L - 1, 2 * L), g2body, 0)

    def pair_body(i, carry):
        k0 = 2 * i
        stage_copy(k0, winbufs[0], sem_a).wait()
        process(k0, winbufs[0])
        stage_copy(k0 + 2, winbufs[0], sem_a).start()
        stage_copy(k0 + 1, winbufs[1], sem_b).wait()
        process(k0 + 1, winbufs[1])
        stage_copy(k0 + 3, winbufs[1], sem_b).start()
        return carry

    lax.fori_loop(0, NWIN // 2, pair_body, 0)
    stage_copy(NWIN, winbufs[1], sem_b).wait()
    avg_stage = pltpu.make_async_copy(avg_t, winbufs[1], sem_b)
    avg_stage.start()
    stage_copy(NWIN - 1, winbufs[0], sem_a).wait()
    process(NWIN - 1, winbufs[0])
    avg_stage.wait()

    def avg_g4(g4, carry):
        copies = []
        for q in range(RING):
            g = g4 * RING + q
            a = aidx_v[pl.ds(g * L, L)]
            dst = wid * B_PER_W + g * L + lane
            for k_e in range(EMBED):
                v = plsc.load_gather(winbufs[1], [_row(k_e), a])
                plsc.store_scatter(rows_q[q], [lane, _row(k_e)], v)
            copies.append(
                pltpu.async_copy(rows_q[q], out_a.at[dst], sem_s))
        for c in copies:
            c.wait()
        return carry

    lax.fori_loop(0, B_PER_W // (RING * L), avg_g4, 0)


def kernel(fans_idx, avgrating_idx, embedding_fans, embedding_avgrating):
    avg_p = jnp.pad(embedding_avgrating.astype(jnp.float32).T,
                    ((0, 0), (0, 1024 - NAVG)))
    out_f, out_a = _lookup(
        fans_idx.astype(jnp.int32),
        avgrating_idx.astype(jnp.int32),
        embedding_fans.T,
        avg_p,
    )
    fans_emb = out_f[:BATCH, :EMBED]
    avg_emb = out_a[:, :EMBED]
    return jnp.concatenate((fans_emb, avg_emb), axis=1)


# speedup vs baseline: 3.8363x; 1.1421x over previous
import functools

import jax
import jax.numpy as jnp
from jax import lax
from jax.experimental import pallas as pl
from jax.experimental.pallas import tpu as pltpu
from jax.experimental.pallas import tpu_sc as plsc

BATCH = 16384
EMBED = 32
NFANS = 1000000
NAVG = 1000
NC = 2
NS = 16
NW = NC * NS
B_PER_W = BATCH // NW
L = 16

NTC = (NFANS + 127) // 128
TC_PER_W = 248
WIN_TC = 8
WIN_COLS = WIN_TC * 128
NWIN = TC_PER_W // WIN_TC
STAGE_CLAMP = (NTC - WIN_TC) * 128

CAP = BATCH + L
PAD_ROWS = NW * L
OUT_ROWS_F = BATCH + PAD_ROWS
RING = 4

_mesh = plsc.VectorSubcoreMesh(core_axis_name="c", subcore_axis_name="s")


def _row(k_e):
    return jnp.full((L,), k_e, jnp.int32)


@functools.partial(
    pl.kernel,
    out_type=(
        jax.ShapeDtypeStruct((OUT_ROWS_F, 128), jnp.float32),
        jax.ShapeDtypeStruct((BATCH, 128), jnp.float32),
    ),
    mesh=_mesh,
    scratch_types=[
        pltpu.VMEM((BATCH,), jnp.int32),
        pltpu.VMEM((B_PER_W,), jnp.int32),
        pltpu.VMEM((CAP,), jnp.int32),
        pltpu.VMEM((CAP,), jnp.int32),
        [pltpu.VMEM((EMBED, WIN_COLS), jnp.float32) for _ in range(2)],
        [pltpu.VMEM((L, 128), jnp.float32) for _ in range(RING)],
        pltpu.SMEM((NWIN + 2,), jnp.int32),
        pltpu.SemaphoreType.DMA,
        pltpu.SemaphoreType.DMA,
        pltpu.SemaphoreType.DMA,
    ],
    compiler_params=pltpu.CompilerParams(
        use_tc_tiling_on_sc=True, needs_layout_passes=False),
)
def _lookup(fans_idx, avg_idx, fans_t, avg_t, out_f, out_a,
            fidx_v, aidx_v, blist, blist2, winbufs,
            rows_q, starts, sem_a, sem_b, sem_s):
    wid = lax.axis_index("s") * NC + lax.axis_index("c")
    lo = wid * TC_PER_W
    lane = lax.iota(jnp.int32, L)

    pltpu.sync_copy(fans_idx, fidx_v)
    pltpu.sync_copy(avg_idx.at[pl.ds(wid * B_PER_W, B_PER_W)], aidx_v)

    def stage_off(k):
        off = jnp.minimum((lo + k * WIN_TC) * 128, STAGE_CLAMP)
        return pl.multiple_of(off, 128)

    def stage_copy(k, buf, sem):
        return pltpu.make_async_copy(
            fans_t.at[:, pl.ds(stage_off(k), WIN_COLS)], buf, sem)

    stage_copy(0, winbufs[0], sem_a).start()
    stage_copy(1, winbufs[1], sem_b).start()

    def scan_body(g4, ptr):
        vals, masks, cnts = [], [], []
        for u in range(4):
            g = 4 * g4 + u
            r = fidx_v[pl.ds(g * L, L)]
            j = lax.shift_right_logical(r, 7)
            m = (j >= lo) & (j < lo + TC_PER_W)
            wk = lax.shift_right_logical(j - lo, 3)
            vals.append((g * L + lane) | (wk << 14))
            masks.append(m)
            cnts.append(plsc.all_reduce_population_count(m)[0])
        tot = cnts[0] + cnts[1] + cnts[2] + cnts[3]
        @pl.when(tot > 0)
        def _():
            p = ptr
            for u in range(4):
                plsc.store_compressed(
                    blist.at[pl.ds(p, L)], vals[u], mask=masks[u])
                p = p + cnts[u]
        return ptr + tot

    cnt = lax.fori_loop(0, BATCH // (4 * L), scan_body, 0)

    starts[0] = 0
    n_groups = lax.div(cnt + L - 1, L)

    def bucket_body(k, ptr2):
        def inner(g, p2):
            v = blist[pl.ds(g * L, L)]
            wk = lax.shift_right_logical(v, 14)
            m = (wk == k) & (g * L + lane < cnt)
            n = plsc.all_reduce_population_count(m)[0]
            @pl.when(n > 0)
            def _():
                plsc.store_compressed(
                    blist2.at[pl.ds(p2, L)], v & (BATCH - 1), mask=m)
            return p2 + n
        ptr2 = lax.fori_loop(0, n_groups, inner, ptr2)
        starts[k + 1] = ptr2
        return ptr2

    lax.fori_loop(0, NWIN, bucket_body, 0)

    def process(k, buf):
        s = starts[k]
        e = starts[k + 1]
        stage = stage_off(k)

        @pl.when(e > s)
        def _():
            def g2body(g2, carry):
                for q in range(2):
                    g = 2 * g2 + q
                    @pl.when(s + g * L < e)
                    def _(g=g, q=q):
                        p = s + g * L + lane
                        m = p < e
                        b = jnp.clip(
                            plsc.load_gather(
                                blist2, [jnp.minimum(p, cnt - 1)]),
                            0, BATCH - 1)
                        r = plsc.load_gather(fidx_v, [b])
                        col = jnp.clip(r - stage, 0, WIN_COLS - 1)
                        dst = jnp.where(m, b, BATCH + wid * L + lane)
                        for k_e in range(EMBED):
                            v = plsc.load_gather(buf, [_row(k_e), col])
                            plsc.store_scatter(
                                rows_q[q], [lane, _row(k_e)], v)
                        pltpu.make_async_copy(
                            rows_q[q], out_f.at[dst], sem_s).start()
                for q in range(2):
                    g = 2 * g2 + q
                    @pl.when(s + g * L < e)
                    def _(q=q):
                        pltpu.make_async_copy(
                            rows_q[q], out_f.at[lane], sem_s).wait()
                return carry

            lax.fori_loop(0, lax.div(e - s + 2 * L - 1, 2 * L), g2body, 0)

    def pair_body(i, carry):
        k0 = 2 * i
        stage_copy(k0, winbufs[0], sem_a).wait()
        process(k0, winbufs[0])
        stage_copy(k0 + 2, winbufs[0], sem_a).start()
        stage_copy(k0 + 1, winbufs[1], sem_b).wait()
        process(k0 + 1, winbufs[1])
        stage_copy(k0 + 3, winbufs[1], sem_b).start()
        return carry

    lax.fori_loop(0, NWIN // 2, pair_body, 0)
    stage_copy(NWIN, winbufs[1], sem_b).wait()
    avg_stage = pltpu.make_async_copy(avg_t, winbufs[1], sem_b)
    avg_stage.start()
    stage_copy(NWIN - 1, winbufs[0], sem_a).wait()
    process(NWIN - 1, winbufs[0])
    avg_stage.wait()

    def avg_g4(g4, carry):
        copies = []
        for q in range(RING):
            g = g4 * RING + q
            a = aidx_v[pl.ds(g * L, L)]
            dst = wid * B_PER_W + g * L + lane
            for k_e in range(EMBED):
                v = plsc.load_gather(winbufs[1], [_row(k_e), a])
                plsc.store_scatter(rows_q[q], [lane, _row(k_e)], v)
            copies.append(
                pltpu.async_copy(rows_q[q], out_a.at[dst], sem_s))
        for c in copies:
            c.wait()
        return carry

    lax.fori_loop(0, B_PER_W // (RING * L), avg_g4, 0)


def kernel(fans_idx, avgrating_idx, embedding_fans, embedding_avgrating):
    avg_p = jnp.pad(embedding_avgrating.astype(jnp.float32).T,
                    ((0, 0), (0, 1024 - NAVG)))
    out_f, out_a = _lookup(
        fans_idx.astype(jnp.int32),
        avgrating_idx.astype(jnp.int32),
        embedding_fans.T,
        avg_p,
    )
    fans_emb = out_f[:BATCH, :EMBED]
    avg_emb = out_a[:, :EMBED]
    return jnp.concatenate((fans_emb, avg_emb), axis=1)
